# baseline (speedup 1.0000x reference)
import jax
import jax.numpy as jnp
from jax import lax
from jax.experimental import pallas as pl
from jax.experimental.pallas import tpu as pltpu
from jax.experimental.pallas import tpu_sc as plsc

L = 16
HW2 = 32
NBUF = 5


def _sc_body(x0_hbm, x1_hbm, bn1_hbm, bn2_hbm, thr_hbm,
             y1_hbm, y2_hbm,
             w_v, thr_v, m1_v, m2_v, *scratch):
    abuf = scratch[0:NBUF]
    bbuf = scratch[NBUF:2 * NBUF]
    sin = scratch[2 * NBUF:3 * NBUF]
    sout = scratch[3 * NBUF:4 * NBUF]

    nc = 2
    wid = lax.axis_index("s") * nc + lax.axis_index("c")
    B, H, W, C = x0_hbm.shape
    cpp = W // HW2
    n_chunks = (B * H * cpp) // (nc * L)
    c0 = wid * n_chunks

    pltpu.sync_copy(thr_hbm, thr_v)
    thr = thr_v[...]
    pltpu.sync_copy(bn1_hbm, w_v)
    for k in range(C // L):
        m1_v[pl.ds(k * L, L)] = jnp.where(
            jnp.abs(w_v[pl.ds(k * L, L)]) >= thr, 1.0, 0.0)
    pltpu.sync_copy(bn2_hbm, w_v)
    for k in range(C // L):
        m2_v[pl.ds(k * L, L)] = jnp.where(
            jnp.abs(w_v[pl.ds(k * L, L)]) >= thr, 1.0, 0.0)

    def _loc(c):
        p = c // cpp
        return p // H, p % H, (c % cpp) * HW2

    def _gather_start(c, s):
        b, h, w0 = _loc(c0 + c)
        pltpu.make_async_copy(
            x0_hbm.at[b, h, pl.ds(w0, HW2)], abuf[s], sin[s]).start()
        pltpu.make_async_copy(
            x1_hbm.at[b, h, pl.ds(w0, HW2)], bbuf[s], sin[s]).start()

    def _gather_wait(s):
        pltpu.make_async_copy(
            x0_hbm.at[0, 0, pl.ds(0, HW2)], abuf[s], sin[s]).wait()
        pltpu.make_async_copy(
            x1_hbm.at[0, 0, pl.ds(0, HW2)], bbuf[s], sin[s]).wait()

    def _scatter_start(c, s):
        b, h, w0 = _loc(c0 + c)
        pltpu.make_async_copy(
            abuf[s], y1_hbm.at[b, h, pl.ds(w0, HW2)], sout[s]).start()
        pltpu.make_async_copy(
            bbuf[s], y2_hbm.at[b, h, pl.ds(w0, HW2)], sout[s]).start()

    def _scatter_wait(s):
        pltpu.make_async_copy(
            abuf[s], y1_hbm.at[0, 0, pl.ds(0, HW2)], sout[s]).wait()
        pltpu.make_async_copy(
            bbuf[s], y2_hbm.at[0, 0, pl.ds(0, HW2)], sout[s]).wait()

    def _compute(s):
        av_ref, bv_ref = abuf[s], bbuf[s]

        def kloop(k):
            sl = pl.ds(k * L, L)
            m1 = m1_v[sl] > 0.5
            m2 = m2_v[sl] > 0.5
            for w in range(HW2):
                av = av_ref[w, sl]
                bv = bv_ref[w, sl]
                av_ref[w, sl] = jnp.where(m1, av, bv)
                bv_ref[w, sl] = jnp.where(m2, bv, av)

        pl.loop(0, C // L)(kloop)

    for s in range(3):
        _gather_start(s, s)

    def step(t):
        for s in range(NBUF):
            c = t * NBUF + s
            snxt = (s + 3) % NBUF

            _gather_wait(s)
            _compute(s)
            _scatter_start(c, s)

            @pl.when(c + 3 < n_chunks)
            def _():
                @pl.when(c >= 2)
                def _():
                    _scatter_wait(snxt)

                _gather_start(c + 3, snxt)

    full_steps = n_chunks // NBUF
    tail = n_chunks - full_steps * NBUF
    pl.loop(0, full_steps)(step)
    for i in range(tail):
        c = full_steps * NBUF + i
        s = c % NBUF
        _gather_wait(s)
        _compute(s)
        _scatter_start(c, s)
    for s in range(NBUF):
        _scatter_wait(s)


def kernel(x0, x1, bn1_weight, bn2_weight, bn_threshold):
    B, C, H, W = x0.shape
    x0t = jnp.transpose(x0, (0, 2, 3, 1))
    x1t = jnp.transpose(x1, (0, 2, 3, 1))
    thr = jnp.full((L,), bn_threshold, dtype=jnp.float32)

    mesh = plsc.VectorSubcoreMesh(core_axis_name="c", subcore_axis_name="s")
    chunk = pltpu.VMEM((HW2, C), jnp.float32)
    run = pl.kernel(
        _sc_body,
        out_type=[
            jax.ShapeDtypeStruct((B, H, W, C), jnp.float32),
            jax.ShapeDtypeStruct((B, H, W, C), jnp.float32),
        ],
        mesh=mesh,
        scratch_types=(
            [pltpu.VMEM((C,), jnp.float32),
             pltpu.VMEM((L,), jnp.float32),
             pltpu.VMEM((C,), jnp.float32),
             pltpu.VMEM((C,), jnp.float32)]
            + [chunk] * (2 * NBUF)
            + [pltpu.SemaphoreType.DMA] * (2 * NBUF)
        ),
        compiler_params=pltpu.CompilerParams(use_tc_tiling_on_sc=True),
    )
    y1t, y2t = run(x0t, x1t, bn1_weight, bn2_weight, thr)
    return (jnp.transpose(y1t, (0, 3, 1, 2)), jnp.transpose(y2t, (0, 3, 1, 2)))
